# Initial kernel scaffold; baseline (speedup 1.0000x reference)
import jax
import jax.numpy as jnp
from jax.experimental import pallas as pl


def kernel(pred_frontal, pred_lateral, source_F, target_F, source_L, target_L, vol_gt_3d, A_inv, b_inv):
    raise NotImplementedError("write your pallas kernel here")


# speedup vs baseline: 1.6744x; 1.6744x over previous
import functools

import jax
import jax.numpy as jnp
from jax import lax
from jax.experimental import pallas as pl
from jax.experimental.pallas import tpu as pltpu
from jax.experimental.pallas import tpu_sc as plsc

H = 128
W = 128
NRAY = H * W
SP = 500
SPP = 512
VOL = 128 * 128 * 128
PAD = 131072
VOLP = VOL + PAD
NTILE = 16
SLAB = VOLP // NTILE
ZCH = 8192
CH = 16
ROWS_PER_TILE = (NRAY * SPP) // NTILE // 128


def _idx_body(srcF_ref, srcL_ref, a_ref, b_ref,
              mF_ref, fx_ref, fy_ref, fz_ref,
              mL_ref, lx_ref, ly_ref, lz_ref,
              outF_ref, outL_ref):
    ki = lax.broadcasted_iota(jnp.int32, (H, SPP), 1)
    kf = ki.astype(jnp.float32)
    ri = lax.broadcasted_iota(jnp.int32, (H, SPP), 0)
    step = jnp.float32(1.0 / (SP - 1))
    tv = kf * step
    dummy = jnp.int32(VOL) + (ki + SPP * (ri & 127))
    a00 = a_ref[0, 0]; a01 = a_ref[0, 1]; a02 = a_ref[0, 2]
    a10 = a_ref[1, 0]; a11 = a_ref[1, 1]; a12 = a_ref[1, 2]
    a20 = a_ref[2, 0]; a21 = a_ref[2, 1]; a22 = a_ref[2, 2]
    b0 = b_ref[0, 0]; b1 = b_ref[0, 1]; b2 = b_ref[0, 2]

    def one_view(src_ref, m_ref, tx_ref, ty_ref, tz_ref, out_ref):
        sx = src_ref[0, 0]; sy = src_ref[0, 1]; sz = src_ref[0, 2]
        dx = tx_ref[0] - sx
        dy = ty_ref[0] - sy
        dz = tz_ref[0] - sz
        ln = jnp.sqrt(dx * dx + dy * dy + dz * dz)
        inv = 1.0 / (ln + jnp.float32(1e-8))
        ux = dx * inv
        uy = dy * inv
        uz = dz * inv
        ts = tv * (ln * jnp.float32(2.5))
        wx = sx + ux * ts
        wy = sy + uy * ts
        wz = sz + uz * ts
        vcx = wx * a00 + wy * a01 + wz * a02 + b0
        vcy = wx * a10 + wy * a11 + wz * a12 + b1
        vcz = wx * a20 + wy * a21 + wz * a22 + b2
        vx = jnp.round(vcx).astype(jnp.int32)
        vy = jnp.round(vcy).astype(jnp.int32)
        vz = jnp.round(vcz).astype(jnp.int32)
        act = m_ref[0] > jnp.float32(0.5)
        valid = (act
                 & (vx >= 0) & (vx < 128)
                 & (vy >= 0) & (vy < 128)
                 & (vz >= 0) & (vz < 128)
                 & (ki < SP))
        lin = (vx * 128 + vy) * 128 + vz
        out_ref[...] = jnp.where(valid, lin, dummy)

    one_view(srcF_ref, mF_ref, fx_ref, fy_ref, fz_ref, outF_ref)
    one_view(srcL_ref, mL_ref, lx_ref, ly_ref, lz_ref, outL_ref)


def _compute_indices(srcF, srcL, a, b, mF, fx, fy, fz, mL, lx, ly, lz):
    col = pl.BlockSpec((1, H, 1), lambda i: (i, 0, 0))
    smem = pl.BlockSpec(memory_space=pltpu.SMEM)
    out_spec = pl.BlockSpec((H, SPP), lambda i: (i, 0))
    return pl.pallas_call(
        _idx_body,
        grid=(H,),
        in_specs=[smem, smem, smem, smem,
                  col, col, col, col,
                  col, col, col, col],
        out_specs=[out_spec, out_spec],
        out_shape=[jax.ShapeDtypeStruct((NRAY, SPP), jnp.int32),
                   jax.ShapeDtypeStruct((NRAY, SPP), jnp.int32)],
    )(srcF, srcL, a, b, mF, fx, fy, fz, mL, lx, ly, lz)


def _scatter_body(idxF_hbm, idxL_hbm, volF_hbm, volL_hbm,
                  idx_v, zeros_v, ones_v, sem):
    c = lax.axis_index("c")
    s = lax.axis_index("s")
    z16 = jnp.zeros((16,), jnp.float32)
    o16 = jnp.ones((16,), jnp.float32)

    def zfill(i, carry):
        zeros_v[pl.ds(i * 16, 16)] = z16
        return carry

    lax.fori_loop(0, ZCH // 16, zfill, 0)
    for i in range(128 // 16):
        ones_v[pl.ds(i * 16, 16)] = o16

    def run(idx_hbm, vol_hbm):
        base = s * SLAB

        def zero_chunk(k, carry):
            pltpu.sync_copy(zeros_v, vol_hbm.at[pl.ds(base + k * ZCH, ZCH)])
            return carry

        lax.fori_loop(0, SLAB // ZCH, zero_chunk, 0)
        plsc.subcore_barrier()

        def scatter_step(j, carry):
            pltpu.sync_copy(idx_hbm.at[s, pl.ds(j * CH, CH)], idx_v)
            cps = [pltpu.async_copy(ones_v, vol_hbm.at[idx_v.at[r]], sem)
                   for r in range(CH)]
            for cp in cps:
                cp.wait()
            return carry

        lax.fori_loop(0, ROWS_PER_TILE // CH, scatter_step, 0)

    @pl.when(c == 0)
    def _():
        run(idxF_hbm, volF_hbm)

    @pl.when(c == 1)
    def _():
        run(idxL_hbm, volL_hbm)


@functools.cache
def _scatter_kernel():
    return functools.partial(
        pl.kernel,
        mesh=plsc.VectorSubcoreMesh(core_axis_name="c", subcore_axis_name="s"),
        out_type=[jax.ShapeDtypeStruct((VOLP,), jnp.float32),
                  jax.ShapeDtypeStruct((VOLP,), jnp.float32)],
        scratch_types=[pltpu.VMEM((CH, 128), jnp.int32),
                       pltpu.VMEM((ZCH,), jnp.float32),
                       pltpu.VMEM((128,), jnp.float32),
                       pltpu.SemaphoreType.DMA],
    )(_scatter_body)


def _scatter_volumes(idxF3, idxL3):
    return _scatter_kernel()(idxF3, idxL3)


def _bce_body(f_ref, l_ref, g_ref, out_ref, acc_ref):
    i = pl.program_id(0)
    n = pl.num_programs(0)

    @pl.when(i == 0)
    def _():
        acc_ref[0] = jnp.float32(0.0)

    ssum = f_ref[...] + l_ref[...]
    p = 1.0 / (1.0 + jnp.exp(-ssum))
    g = g_ref[...]
    term = g * jnp.log(p) + (1.0 - g) * jnp.log(1.0 - p)
    acc_ref[0] = acc_ref[0] + jnp.sum(term)

    @pl.when(i == n - 1)
    def _():
        out_ref[0, 0] = acc_ref[0] * jnp.float32(-1.0 / VOL)


def _bce_loss(volF, volL, gt):
    blk = pl.BlockSpec((1024, 128), lambda i: (i, 0))
    return pl.pallas_call(
        _bce_body,
        grid=(16,),
        in_specs=[blk, blk, blk],
        out_specs=pl.BlockSpec(memory_space=pltpu.SMEM),
        out_shape=jax.ShapeDtypeStruct((1, 1), jnp.float32),
        scratch_shapes=[pltpu.SMEM((1,), jnp.float32)],
    )(volF, volL, gt)


def kernel(pred_frontal, pred_lateral, source_F, target_F, source_L,
           target_L, vol_gt_3d, A_inv, b_inv):
    mF = pred_frontal[0, 0].reshape(H, W, 1)
    mL = pred_lateral[0, 0].reshape(H, W, 1)
    fx = target_F[0, :, :, 0].reshape(H, W, 1)
    fy = target_F[0, :, :, 1].reshape(H, W, 1)
    fz = target_F[0, :, :, 2].reshape(H, W, 1)
    lx = target_L[0, :, :, 0].reshape(H, W, 1)
    ly = target_L[0, :, :, 1].reshape(H, W, 1)
    lz = target_L[0, :, :, 2].reshape(H, W, 1)
    b2d = b_inv.reshape(1, 3)

    idxF, idxL = _compute_indices(source_F, source_L, A_inv, b2d,
                                  mF, fx, fy, fz, mL, lx, ly, lz)
    idxF3 = idxF.reshape(NTILE, ROWS_PER_TILE, 128)
    idxL3 = idxL.reshape(NTILE, ROWS_PER_TILE, 128)
    volF, volL = _scatter_volumes(idxF3, idxL3)
    loss = _bce_loss(volF.reshape(VOLP // 128, 128),
                     volL.reshape(VOLP // 128, 128),
                     vol_gt_3d.reshape(NRAY, 128))
    return loss[0, 0]


# speedup vs baseline: 58.0866x; 34.6906x over previous
import functools

import jax
import jax.numpy as jnp
from jax import lax
from jax.experimental import pallas as pl
from jax.experimental.pallas import tpu as pltpu
from jax.experimental.pallas import tpu_sc as plsc

H = 128
W = 128
NRAY = H * W
SP = 500
SPP = 512
VOL = 128 * 128 * 128
HALF = VOL // 2
SINK = 256
NTILE = 16
ZCH = 8192
CH = 16
ROWS_PER_TILE = (NRAY * SPP) // NTILE // 128


def _idx_body(srcF_ref, srcL_ref, a_ref, b_ref,
              mF_ref, fx_ref, fy_ref, fz_ref,
              mL_ref, lx_ref, ly_ref, lz_ref,
              loF_ref, hiF_ref, loL_ref, hiL_ref, flgF_ref, flgL_ref):
    ki = lax.broadcasted_iota(jnp.int32, (H, SPP), 1)
    kf = ki.astype(jnp.float32)
    ri = lax.broadcasted_iota(jnp.int32, (H, SPP), 0)
    step = jnp.float32(1.0 / (SP - 1))
    tv = kf * step
    dummy = jnp.int32(HALF) + ((ki + ri) & (SINK - 1))
    a00 = a_ref[0, 0]; a01 = a_ref[0, 1]; a02 = a_ref[0, 2]
    a10 = a_ref[1, 0]; a11 = a_ref[1, 1]; a12 = a_ref[1, 2]
    a20 = a_ref[2, 0]; a21 = a_ref[2, 1]; a22 = a_ref[2, 2]
    b0 = b_ref[0, 0]; b1 = b_ref[0, 1]; b2 = b_ref[0, 2]

    def one_view(src_ref, m_ref, tx_ref, ty_ref, tz_ref,
                 lo_ref, hi_ref, flg_ref):
        sx = src_ref[0, 0]; sy = src_ref[0, 1]; sz = src_ref[0, 2]
        dx = tx_ref[0] - sx
        dy = ty_ref[0] - sy
        dz = tz_ref[0] - sz
        ln = jnp.sqrt(dx * dx + dy * dy + dz * dz)
        inv = 1.0 / (ln + jnp.float32(1e-8))
        ux = dx * inv
        uy = dy * inv
        uz = dz * inv
        ts = tv * (ln * jnp.float32(2.5))
        wx = sx + ux * ts
        wy = sy + uy * ts
        wz = sz + uz * ts
        vcx = wx * a00 + wy * a01 + wz * a02 + b0
        vcy = wx * a10 + wy * a11 + wz * a12 + b1
        vcz = wx * a20 + wy * a21 + wz * a22 + b2
        vx = jnp.round(vcx).astype(jnp.int32)
        vy = jnp.round(vcy).astype(jnp.int32)
        vz = jnp.round(vcz).astype(jnp.int32)
        act = m_ref[0] > jnp.float32(0.5)
        valid = (act
                 & (vx >= 0) & (vx < 128)
                 & (vy >= 0) & (vy < 128)
                 & (vz >= 0) & (vz < 128)
                 & (ki < SP))
        lin = (vx * 128 + vy) * 128 + vz
        lo_ref[...] = jnp.where(valid & (lin < HALF), lin, dummy)
        hi_ref[...] = jnp.where(valid & (lin >= HALF), lin - HALF, dummy)
        flg_ref[...] = jnp.broadcast_to(
            jnp.where(act, jnp.int32(1), jnp.int32(0)), (H, 4))

    one_view(srcF_ref, mF_ref, fx_ref, fy_ref, fz_ref,
             loF_ref, hiF_ref, flgF_ref)
    one_view(srcL_ref, mL_ref, lx_ref, ly_ref, lz_ref,
             loL_ref, hiL_ref, flgL_ref)


def _compute_indices(srcF, srcL, a, b, mF, fx, fy, fz, mL, lx, ly, lz):
    col = pl.BlockSpec((1, H, 1), lambda i: (i, 0, 0))
    smem = pl.BlockSpec(memory_space=pltpu.SMEM)
    out_spec = pl.BlockSpec((H, SPP), lambda i: (i, 0))
    flg_spec = pl.BlockSpec((H, 4), lambda i: (i, 0))
    return pl.pallas_call(
        _idx_body,
        grid=(H,),
        in_specs=[smem, smem, smem, smem,
                  col, col, col, col,
                  col, col, col, col],
        out_specs=[out_spec, out_spec, out_spec, out_spec,
                   flg_spec, flg_spec],
        out_shape=[jax.ShapeDtypeStruct((NRAY, SPP), jnp.int32),
                   jax.ShapeDtypeStruct((NRAY, SPP), jnp.int32),
                   jax.ShapeDtypeStruct((NRAY, SPP), jnp.int32),
                   jax.ShapeDtypeStruct((NRAY, SPP), jnp.int32),
                   jax.ShapeDtypeStruct((NRAY, 4), jnp.int32),
                   jax.ShapeDtypeStruct((NRAY, 4), jnp.int32)],
    )(srcF, srcL, a, b, mF, fx, fy, fz, mL, lx, ly, lz)


ZSLAB = (HALF + SINK) // NTILE
OSLAB = HALF // NTILE


def _scatter_body(loF_hbm, hiF_hbm, loL_hbm, hiL_hbm, flgF_hbm, flgL_hbm,
                  volF_hbm, volL_hbm,
                  idx_v, flg_v, zeros_v, ones_v, stage_v, vol_sh, sem):
    c = lax.axis_index("c")
    s = lax.axis_index("s")
    z16 = jnp.zeros((16,), jnp.float32)
    o16 = jnp.ones((16,), jnp.float32)

    def zfill(i, carry):
        zeros_v[pl.ds(i * 16, 16)] = z16
        return carry

    lax.fori_loop(0, ZCH // 16, zfill, 0)
    for i in range(128 // 16):
        ones_v[pl.ds(i * 16, 16)] = o16

    def one_pass(idx_hbm, flg_hbm, vol_hbm, out_base):
        zbase = s * ZSLAB

        def zero_chunk(k, carry):
            pltpu.sync_copy(zeros_v, vol_sh.at[pl.ds(zbase + k * ZCH, ZCH)])
            return carry

        lax.fori_loop(0, ZSLAB // ZCH, zero_chunk, 0)
        rem = ZSLAB - (ZSLAB // ZCH) * ZCH
        if rem:
            pltpu.sync_copy(zeros_v.at[pl.ds(0, rem)],
                            vol_sh.at[pl.ds(zbase + (ZSLAB // ZCH) * ZCH, rem)])
        plsc.subcore_barrier()

        def scatter_step(j, carry):
            pltpu.sync_copy(idx_hbm.at[s, pl.ds(j * CH, CH)], idx_v)
            pltpu.sync_copy(flg_hbm.at[s, pl.ds(j * CH, CH)], flg_v)
            fv = flg_v[...]
            for r in range(CH):
                @pl.when(fv[r] != 0)
                def _():
                    pltpu.async_copy(ones_v, vol_sh.at[idx_v.at[r]], sem)
            for r in range(CH):
                @pl.when(fv[r] != 0)
                def _():
                    pltpu.make_async_copy(
                        ones_v, vol_sh.at[idx_v.at[r]], sem).wait()
            return carry

        lax.fori_loop(0, ROWS_PER_TILE // CH, scatter_step, 0)
        plsc.subcore_barrier()
        obase = s * OSLAB
        for k in range(OSLAB // ZCH):
            pltpu.sync_copy(vol_sh.at[pl.ds(obase + k * ZCH, ZCH)],
                            stage_v)
            pltpu.sync_copy(stage_v,
                            vol_hbm.at[pl.ds(out_base + obase + k * ZCH, ZCH)])
        plsc.subcore_barrier()

    def run(lo_hbm, hi_hbm, flg_hbm, vol_hbm):
        one_pass(lo_hbm, flg_hbm, vol_hbm, 0)
        one_pass(hi_hbm, flg_hbm, vol_hbm, HALF)

    @pl.when(c == 0)
    def _():
        run(loF_hbm, hiF_hbm, flgF_hbm, volF_hbm)

    @pl.when(c == 1)
    def _():
        run(loL_hbm, hiL_hbm, flgL_hbm, volL_hbm)


@functools.cache
def _scatter_kernel():
    return functools.partial(
        pl.kernel,
        mesh=plsc.VectorSubcoreMesh(core_axis_name="c", subcore_axis_name="s"),
        out_type=[jax.ShapeDtypeStruct((VOL,), jnp.float32),
                  jax.ShapeDtypeStruct((VOL,), jnp.float32)],
        scratch_types=[pltpu.VMEM((CH, 128), jnp.int32),
                       pltpu.VMEM((CH,), jnp.int32),
                       pltpu.VMEM((ZCH,), jnp.float32),
                       pltpu.VMEM((128,), jnp.float32),
                       pltpu.VMEM((ZCH,), jnp.float32),
                       pltpu.VMEM_SHARED((HALF + SINK,), jnp.float32),
                       pltpu.SemaphoreType.DMA],
    )(_scatter_body)


def _scatter_volumes(loF3, hiF3, loL3, hiL3, flgF2, flgL2):
    return _scatter_kernel()(loF3, hiF3, loL3, hiL3, flgF2, flgL2)


def _bce_body(f_ref, l_ref, g_ref, out_ref, acc_ref):
    i = pl.program_id(0)
    n = pl.num_programs(0)

    @pl.when(i == 0)
    def _():
        acc_ref[0] = jnp.float32(0.0)

    ssum = f_ref[...] + l_ref[...]
    p = 1.0 / (1.0 + jnp.exp(-ssum))
    g = g_ref[...]
    term = g * jnp.log(p) + (1.0 - g) * jnp.log(1.0 - p)
    acc_ref[0] = acc_ref[0] + jnp.sum(term)

    @pl.when(i == n - 1)
    def _():
        out_ref[0, 0] = acc_ref[0] * jnp.float32(-1.0 / VOL)


def _bce_loss(volF, volL, gt):
    blk = pl.BlockSpec((1024, 128), lambda i: (i, 0))
    return pl.pallas_call(
        _bce_body,
        grid=(16,),
        in_specs=[blk, blk, blk],
        out_specs=pl.BlockSpec(memory_space=pltpu.SMEM),
        out_shape=jax.ShapeDtypeStruct((1, 1), jnp.float32),
        scratch_shapes=[pltpu.SMEM((1,), jnp.float32)],
    )(volF, volL, gt)


def kernel(pred_frontal, pred_lateral, source_F, target_F, source_L,
           target_L, vol_gt_3d, A_inv, b_inv):
    mF = pred_frontal[0, 0].reshape(H, W, 1)
    mL = pred_lateral[0, 0].reshape(H, W, 1)
    fx = target_F[0, :, :, 0].reshape(H, W, 1)
    fy = target_F[0, :, :, 1].reshape(H, W, 1)
    fz = target_F[0, :, :, 2].reshape(H, W, 1)
    lx = target_L[0, :, :, 0].reshape(H, W, 1)
    ly = target_L[0, :, :, 1].reshape(H, W, 1)
    lz = target_L[0, :, :, 2].reshape(H, W, 1)
    b2d = b_inv.reshape(1, 3)

    loF, hiF, loL, hiL, flgF, flgL = _compute_indices(
        source_F, source_L, A_inv, b2d,
        mF, fx, fy, fz, mL, lx, ly, lz)
    r3 = lambda a: a.reshape(NTILE, ROWS_PER_TILE, 128)
    volF, volL = _scatter_volumes(
        r3(loF), r3(hiF), r3(loL), r3(hiL),
        flgF.reshape(NTILE, ROWS_PER_TILE),
        flgL.reshape(NTILE, ROWS_PER_TILE))
    loss = _bce_loss(volF.reshape(NRAY, 128),
                     volL.reshape(NRAY, 128),
                     vol_gt_3d.reshape(NRAY, 128))
    return loss[0, 0]


# speedup vs baseline: 58.1453x; 1.0010x over previous
import functools

import jax
import jax.numpy as jnp
from jax import lax
from jax.experimental import pallas as pl
from jax.experimental.pallas import tpu as pltpu
from jax.experimental.pallas import tpu_sc as plsc

H = 128
W = 128
NRAY = H * W
SP = 500
SPP = 512
VOL = 128 * 128 * 128
HALF = VOL // 2
SINK = 256
NTILE = 16
ZCH = 8192
CH = 16
ROWS_PER_TILE = (NRAY * SPP) // NTILE // 128


def _idx_body(srcF_ref, srcL_ref, a_ref, b_ref,
              mF_ref, fx_ref, fy_ref, fz_ref,
              mL_ref, lx_ref, ly_ref, lz_ref,
              loF_ref, hiF_ref, loL_ref, hiL_ref, flgF_ref, flgL_ref):
    ki = lax.broadcasted_iota(jnp.int32, (H, SPP), 1)
    kf = ki.astype(jnp.float32)
    ri = lax.broadcasted_iota(jnp.int32, (H, SPP), 0)
    step = jnp.float32(1.0 / (SP - 1))
    tv = kf * step
    dummy = jnp.int32(HALF) + ((ki + ri) & (SINK - 1))
    a00 = a_ref[0, 0]; a01 = a_ref[0, 1]; a02 = a_ref[0, 2]
    a10 = a_ref[1, 0]; a11 = a_ref[1, 1]; a12 = a_ref[1, 2]
    a20 = a_ref[2, 0]; a21 = a_ref[2, 1]; a22 = a_ref[2, 2]
    b0 = b_ref[0, 0]; b1 = b_ref[0, 1]; b2 = b_ref[0, 2]

    def one_view(src_ref, m_ref, tx_ref, ty_ref, tz_ref,
                 lo_ref, hi_ref, flg_ref):
        sx = src_ref[0, 0]; sy = src_ref[0, 1]; sz = src_ref[0, 2]
        dx = tx_ref[0] - sx
        dy = ty_ref[0] - sy
        dz = tz_ref[0] - sz
        ln = jnp.sqrt(dx * dx + dy * dy + dz * dz)
        lden = ln + jnp.float32(1e-8)
        ux = dx / lden
        uy = dy / lden
        uz = dz / lden
        ts = tv * (ln * jnp.float32(2.5))
        wx = sx + ux * ts
        wy = sy + uy * ts
        wz = sz + uz * ts
        vcx = wx * a00 + wy * a01 + wz * a02 + b0
        vcy = wx * a10 + wy * a11 + wz * a12 + b1
        vcz = wx * a20 + wy * a21 + wz * a22 + b2
        vx = jnp.round(vcx).astype(jnp.int32)
        vy = jnp.round(vcy).astype(jnp.int32)
        vz = jnp.round(vcz).astype(jnp.int32)
        act = m_ref[0] > jnp.float32(0.5)
        valid = (act
                 & (vx >= 0) & (vx < 128)
                 & (vy >= 0) & (vy < 128)
                 & (vz >= 0) & (vz < 128)
                 & (ki < SP))
        lin = (vx * 128 + vy) * 128 + vz
        lo_ref[...] = jnp.where(valid & (lin < HALF), lin, dummy)
        hi_ref[...] = jnp.where(valid & (lin >= HALF), lin - HALF, dummy)
        flg_ref[...] = jnp.broadcast_to(
            jnp.where(act, jnp.int32(1), jnp.int32(0)), (H, 4))

    one_view(srcF_ref, mF_ref, fx_ref, fy_ref, fz_ref,
             loF_ref, hiF_ref, flgF_ref)
    one_view(srcL_ref, mL_ref, lx_ref, ly_ref, lz_ref,
             loL_ref, hiL_ref, flgL_ref)


def _compute_indices(srcF, srcL, a, b, mF, fx, fy, fz, mL, lx, ly, lz):
    col = pl.BlockSpec((1, H, 1), lambda i: (i, 0, 0))
    smem = pl.BlockSpec(memory_space=pltpu.SMEM)
    out_spec = pl.BlockSpec((H, SPP), lambda i: (i, 0))
    flg_spec = pl.BlockSpec((H, 4), lambda i: (i, 0))
    return pl.pallas_call(
        _idx_body,
        grid=(H,),
        in_specs=[smem, smem, smem, smem,
                  col, col, col, col,
                  col, col, col, col],
        out_specs=[out_spec, out_spec, out_spec, out_spec,
                   flg_spec, flg_spec],
        out_shape=[jax.ShapeDtypeStruct((NRAY, SPP), jnp.int32),
                   jax.ShapeDtypeStruct((NRAY, SPP), jnp.int32),
                   jax.ShapeDtypeStruct((NRAY, SPP), jnp.int32),
                   jax.ShapeDtypeStruct((NRAY, SPP), jnp.int32),
                   jax.ShapeDtypeStruct((NRAY, 4), jnp.int32),
                   jax.ShapeDtypeStruct((NRAY, 4), jnp.int32)],
    )(srcF, srcL, a, b, mF, fx, fy, fz, mL, lx, ly, lz)


ZSLAB = (HALF + SINK) // NTILE
OSLAB = HALF // NTILE


def _scatter_body(loF_hbm, hiF_hbm, loL_hbm, hiL_hbm, flgF_hbm, flgL_hbm,
                  volF_hbm, volL_hbm,
                  idx_v, flg_v, zeros_v, ones_v, stage_v, vol_sh, sem):
    c = lax.axis_index("c")
    s = lax.axis_index("s")
    z16 = jnp.zeros((16,), jnp.float32)
    o16 = jnp.ones((16,), jnp.float32)

    def zfill(i, carry):
        zeros_v[pl.ds(i * 16, 16)] = z16
        return carry

    lax.fori_loop(0, ZCH // 16, zfill, 0)
    for i in range(128 // 16):
        ones_v[pl.ds(i * 16, 16)] = o16

    def one_pass(idx_hbm, flg_hbm, vol_hbm, out_base):
        zbase = s * ZSLAB

        def zero_chunk(k, carry):
            pltpu.sync_copy(zeros_v, vol_sh.at[pl.ds(zbase + k * ZCH, ZCH)])
            return carry

        lax.fori_loop(0, ZSLAB // ZCH, zero_chunk, 0)
        rem = ZSLAB - (ZSLAB // ZCH) * ZCH
        if rem:
            pltpu.sync_copy(zeros_v.at[pl.ds(0, rem)],
                            vol_sh.at[pl.ds(zbase + (ZSLAB // ZCH) * ZCH, rem)])
        plsc.subcore_barrier()

        def scatter_step(j, carry):
            pltpu.sync_copy(idx_hbm.at[s, pl.ds(j * CH, CH)], idx_v)
            pltpu.sync_copy(flg_hbm.at[s, pl.ds(j * CH, CH)], flg_v)
            fv = flg_v[...]
            for r in range(CH):
                @pl.when(fv[r] != 0)
                def _():
                    pltpu.async_copy(ones_v, vol_sh.at[idx_v.at[r]], sem)
            for r in range(CH):
                @pl.when(fv[r] != 0)
                def _():
                    pltpu.make_async_copy(
                        ones_v, vol_sh.at[idx_v.at[r]], sem).wait()
            return carry

        lax.fori_loop(0, ROWS_PER_TILE // CH, scatter_step, 0)
        plsc.subcore_barrier()
        obase = s * OSLAB
        for k in range(OSLAB // ZCH):
            pltpu.sync_copy(vol_sh.at[pl.ds(obase + k * ZCH, ZCH)],
                            stage_v)
            pltpu.sync_copy(stage_v,
                            vol_hbm.at[pl.ds(out_base + obase + k * ZCH, ZCH)])
        plsc.subcore_barrier()

    def run(lo_hbm, hi_hbm, flg_hbm, vol_hbm):
        one_pass(lo_hbm, flg_hbm, vol_hbm, 0)
        one_pass(hi_hbm, flg_hbm, vol_hbm, HALF)

    @pl.when(c == 0)
    def _():
        run(loF_hbm, hiF_hbm, flgF_hbm, volF_hbm)

    @pl.when(c == 1)
    def _():
        run(loL_hbm, hiL_hbm, flgL_hbm, volL_hbm)


@functools.cache
def _scatter_kernel():
    return functools.partial(
        pl.kernel,
        mesh=plsc.VectorSubcoreMesh(core_axis_name="c", subcore_axis_name="s"),
        out_type=[jax.ShapeDtypeStruct((VOL,), jnp.float32),
                  jax.ShapeDtypeStruct((VOL,), jnp.float32)],
        scratch_types=[pltpu.VMEM((CH, 128), jnp.int32),
                       pltpu.VMEM((CH,), jnp.int32),
                       pltpu.VMEM((ZCH,), jnp.float32),
                       pltpu.VMEM((128,), jnp.float32),
                       pltpu.VMEM((ZCH,), jnp.float32),
                       pltpu.VMEM_SHARED((HALF + SINK,), jnp.float32),
                       pltpu.SemaphoreType.DMA],
    )(_scatter_body)


def _scatter_volumes(loF3, hiF3, loL3, hiL3, flgF2, flgL2):
    return _scatter_kernel()(loF3, hiF3, loL3, hiL3, flgF2, flgL2)


def _bce_body(f_ref, l_ref, g_ref, out_ref, acc_ref):
    i = pl.program_id(0)
    n = pl.num_programs(0)

    @pl.when(i == 0)
    def _():
        acc_ref[0] = jnp.float32(0.0)

    ssum = f_ref[...] + l_ref[...]
    p = 1.0 / (1.0 + jnp.exp(-ssum))
    g = g_ref[...]
    term = g * jnp.log(p) + (1.0 - g) * jnp.log(1.0 - p)
    acc_ref[0] = acc_ref[0] + jnp.sum(term)

    @pl.when(i == n - 1)
    def _():
        out_ref[0, 0] = acc_ref[0] * jnp.float32(-1.0 / VOL)


def _bce_loss(volF, volL, gt):
    blk = pl.BlockSpec((1024, 128), lambda i: (i, 0))
    return pl.pallas_call(
        _bce_body,
        grid=(16,),
        in_specs=[blk, blk, blk],
        out_specs=pl.BlockSpec(memory_space=pltpu.SMEM),
        out_shape=jax.ShapeDtypeStruct((1, 1), jnp.float32),
        scratch_shapes=[pltpu.SMEM((1,), jnp.float32)],
    )(volF, volL, gt)


def kernel(pred_frontal, pred_lateral, source_F, target_F, source_L,
           target_L, vol_gt_3d, A_inv, b_inv):
    mF = pred_frontal[0, 0].reshape(H, W, 1)
    mL = pred_lateral[0, 0].reshape(H, W, 1)
    fx = target_F[0, :, :, 0].reshape(H, W, 1)
    fy = target_F[0, :, :, 1].reshape(H, W, 1)
    fz = target_F[0, :, :, 2].reshape(H, W, 1)
    lx = target_L[0, :, :, 0].reshape(H, W, 1)
    ly = target_L[0, :, :, 1].reshape(H, W, 1)
    lz = target_L[0, :, :, 2].reshape(H, W, 1)
    b2d = b_inv.reshape(1, 3)

    loF, hiF, loL, hiL, flgF, flgL = _compute_indices(
        source_F, source_L, A_inv, b2d,
        mF, fx, fy, fz, mL, lx, ly, lz)
    r3 = lambda a: a.reshape(NTILE, ROWS_PER_TILE, 128)
    volF, volL = _scatter_volumes(
        r3(loF), r3(hiF), r3(loL), r3(hiL),
        flgF.reshape(NTILE, ROWS_PER_TILE),
        flgL.reshape(NTILE, ROWS_PER_TILE))
    loss = _bce_loss(volF.reshape(NRAY, 128),
                     volL.reshape(NRAY, 128),
                     vol_gt_3d.reshape(NRAY, 128))
    return loss[0, 0]
